# baseline (device time: 121747 ns/iter reference)
import jax
import jax.numpy as jnp
from jax import lax
from jax.experimental import pallas as pl
from jax.experimental.pallas import tpu as pltpu

C = 16


def kernel(x):
    _, m, n2 = x.shape
    n = n2 // 2
    half_m = m // 2
    ch = half_m // C

    def body(
        x_ref,
        out_ref,
        comm_ref,
        lin_ref,
        part_ref,
        y_send,
        y_recv,
        x_send,
        x_recv,
        lin_sems,
        copyout_sems,
    ):
        my_x = lax.axis_index("x")
        my_y = lax.axis_index("y")
        y_peer = (my_x, 1 - my_y)
        x_peer = (1 - my_x, my_y)

        barrier = pltpu.get_barrier_semaphore()
        for p in (y_peer, x_peer):
            pl.semaphore_signal(
                barrier, inc=1, device_id=p, device_id_type=pl.DeviceIdType.MESH
            )
        pl.semaphore_wait(barrier, 2)

        row0 = my_x * half_m
        my_col0 = my_y * n
        peer_col0 = (1 - my_y) * n

        y_rdmas = []
        x_rdmas = []
        lin_dmas = []
        out_dmas = []

        for c in range(C):
            r = row0 + c * ch
            dma = pltpu.make_async_copy(
                x_ref.at[0, pl.ds(r, ch), pl.ds(my_col0, n)],
                lin_ref.at[c],
                lin_sems.at[c],
            )
            dma.start()
            lin_dmas.append(dma)
            rdma = pltpu.make_async_remote_copy(
                src_ref=x_ref.at[0, pl.ds(r, ch), pl.ds(peer_col0, n)],
                dst_ref=comm_ref.at[c],
                send_sem=y_send.at[c],
                recv_sem=y_recv.at[c],
                device_id=y_peer,
                device_id_type=pl.DeviceIdType.MESH,
            )
            rdma.start()
            y_rdmas.append(rdma)

        for c in range(C):
            r = row0 + c * ch
            y_rdmas[c].wait_recv()
            lin_dmas[c].wait()
            part_ref[c] = lin_ref[c] + comm_ref[c]
            rdma = pltpu.make_async_remote_copy(
                src_ref=part_ref.at[c],
                dst_ref=out_ref.at[pl.ds(r, ch), :],
                send_sem=x_send.at[c],
                recv_sem=x_recv.at[c],
                device_id=x_peer,
                device_id_type=pl.DeviceIdType.MESH,
            )
            rdma.start()
            x_rdmas.append(rdma)
            dma = pltpu.make_async_copy(
                part_ref.at[c],
                out_ref.at[pl.ds(r, ch), :],
                copyout_sems.at[c],
            )
            dma.start()
            out_dmas.append(dma)

        for c in range(C):
            y_rdmas[c].wait_send()
            x_rdmas[c].wait_send()
            x_rdmas[c].wait_recv()
            out_dmas[c].wait()

    return pl.pallas_call(
        body,
        out_shape=jax.ShapeDtypeStruct((m, n), x.dtype),
        in_specs=[pl.BlockSpec(memory_space=pl.ANY)],
        out_specs=pl.BlockSpec(memory_space=pltpu.VMEM),
        scratch_shapes=[
            pltpu.VMEM((C, ch, n), x.dtype),
            pltpu.VMEM((C, ch, n), x.dtype),
            pltpu.VMEM((C, ch, n), x.dtype),
            pltpu.SemaphoreType.DMA((C,)),
            pltpu.SemaphoreType.DMA((C,)),
            pltpu.SemaphoreType.DMA((C,)),
            pltpu.SemaphoreType.DMA((C,)),
            pltpu.SemaphoreType.DMA((C,)),
            pltpu.SemaphoreType.DMA((C,)),
        ],
        compiler_params=pltpu.CompilerParams(
            collective_id=0, vmem_limit_bytes=48 * 1024 * 1024
        ),
    )(x)


# device time: 116214 ns/iter; 1.0476x vs baseline; 1.0476x over previous
import jax
import jax.numpy as jnp
from jax import lax
from jax.experimental import pallas as pl
from jax.experimental.pallas import tpu as pltpu

C = 16


def kernel(x):
    _, m, n2 = x.shape
    n = n2 // 2
    half_m = m // 2
    ch = half_m // C

    def body(
        x_ref,
        out_ref,
        comm_ref,
        lin_ref,
        part_ref,
        y_send,
        y_recv,
        x_send,
        x_recv,
        lin_sems,
        copyout_sems,
    ):
        my_x = lax.axis_index("x")
        my_y = lax.axis_index("y")
        y_peer = (my_x, 1 - my_y)
        x_peer = (1 - my_x, my_y)

        barrier = pltpu.get_barrier_semaphore()
        for p in (y_peer, x_peer):
            pl.semaphore_signal(
                barrier, inc=1, device_id=p, device_id_type=pl.DeviceIdType.MESH
            )
        pl.semaphore_wait(barrier, 2)

        row0 = my_x * half_m
        my_col0 = my_y * n
        peer_col0 = (1 - my_y) * n

        y_rdmas = []
        x_rdmas = []
        lin_dmas = []
        out_dmas = []

        for c in range(C):
            r = row0 + c * ch
            dma = pltpu.make_async_copy(
                x_ref.at[0, pl.ds(r, ch), pl.ds(my_col0, n)],
                lin_ref.at[c],
                lin_sems.at[c],
            )
            dma.start()
            lin_dmas.append(dma)
            rdma = pltpu.make_async_remote_copy(
                src_ref=x_ref.at[0, pl.ds(r, ch), pl.ds(peer_col0, n)],
                dst_ref=comm_ref.at[c],
                send_sem=y_send.at[c],
                recv_sem=y_recv.at[c],
                device_id=y_peer,
                device_id_type=pl.DeviceIdType.MESH,
            )
            rdma.start()
            y_rdmas.append(rdma)

        for c in range(C):
            r = row0 + c * ch
            y_rdmas[c].wait_recv()
            lin_dmas[c].wait()
            part_ref[c] = lin_ref[c] + comm_ref[c]
            rdma = pltpu.make_async_remote_copy(
                src_ref=part_ref.at[c],
                dst_ref=out_ref.at[pl.ds(r, ch), :],
                send_sem=x_send.at[c],
                recv_sem=x_recv.at[c],
                device_id=x_peer,
                device_id_type=pl.DeviceIdType.MESH,
            )
            rdma.start()
            x_rdmas.append(rdma)
            dma = pltpu.make_async_copy(
                part_ref.at[c],
                out_ref.at[pl.ds(r, ch), :],
                copyout_sems.at[c],
            )
            dma.start()
            out_dmas.append(dma)

        for c in range(C):
            y_rdmas[c].wait_send()
            x_rdmas[c].wait_send()
            x_rdmas[c].wait_recv()
            out_dmas[c].wait()

    return pl.pallas_call(
        body,
        out_shape=jax.ShapeDtypeStruct((m, n), x.dtype),
        in_specs=[pl.BlockSpec(memory_space=pl.ANY)],
        out_specs=pl.BlockSpec(memory_space=pl.ANY),
        scratch_shapes=[
            pltpu.VMEM((C, ch, n), x.dtype),
            pltpu.VMEM((C, ch, n), x.dtype),
            pltpu.VMEM((C, ch, n), x.dtype),
            pltpu.SemaphoreType.DMA((C,)),
            pltpu.SemaphoreType.DMA((C,)),
            pltpu.SemaphoreType.DMA((C,)),
            pltpu.SemaphoreType.DMA((C,)),
            pltpu.SemaphoreType.DMA((C,)),
            pltpu.SemaphoreType.DMA((C,)),
        ],
        compiler_params=pltpu.CompilerParams(
            collective_id=0, vmem_limit_bytes=48 * 1024 * 1024
        ),
    )(x)


# device time: 114197 ns/iter; 1.0661x vs baseline; 1.0177x over previous
import jax
import jax.numpy as jnp
from jax import lax
from jax.experimental import pallas as pl
from jax.experimental.pallas import tpu as pltpu

C = 32


def kernel(x):
    _, m, n2 = x.shape
    n = n2 // 2
    half_m = m // 2
    ch = half_m // C

    def body(
        x_ref,
        out_ref,
        comm_ref,
        lin_ref,
        part_ref,
        y_send,
        y_recv,
        x_send,
        x_recv,
        lin_sems,
        copyout_sems,
    ):
        my_x = lax.axis_index("x")
        my_y = lax.axis_index("y")
        y_peer = (my_x, 1 - my_y)
        x_peer = (1 - my_x, my_y)

        barrier = pltpu.get_barrier_semaphore()
        for p in (y_peer, x_peer):
            pl.semaphore_signal(
                barrier, inc=1, device_id=p, device_id_type=pl.DeviceIdType.MESH
            )
        pl.semaphore_wait(barrier, 2)

        row0 = my_x * half_m
        my_col0 = my_y * n
        peer_col0 = (1 - my_y) * n

        y_rdmas = []
        x_rdmas = []
        lin_dmas = []
        out_dmas = []

        for c in range(C):
            r = row0 + c * ch
            dma = pltpu.make_async_copy(
                x_ref.at[0, pl.ds(r, ch), pl.ds(my_col0, n)],
                lin_ref.at[c],
                lin_sems.at[c],
            )
            dma.start()
            lin_dmas.append(dma)
            rdma = pltpu.make_async_remote_copy(
                src_ref=x_ref.at[0, pl.ds(r, ch), pl.ds(peer_col0, n)],
                dst_ref=comm_ref.at[c],
                send_sem=y_send.at[c],
                recv_sem=y_recv.at[c],
                device_id=y_peer,
                device_id_type=pl.DeviceIdType.MESH,
            )
            rdma.start()
            y_rdmas.append(rdma)

        for c in range(C):
            r = row0 + c * ch
            y_rdmas[c].wait_recv()
            lin_dmas[c].wait()
            part_ref[c] = lin_ref[c] + comm_ref[c]
            rdma = pltpu.make_async_remote_copy(
                src_ref=part_ref.at[c],
                dst_ref=out_ref.at[pl.ds(r, ch), :],
                send_sem=x_send.at[c],
                recv_sem=x_recv.at[c],
                device_id=x_peer,
                device_id_type=pl.DeviceIdType.MESH,
            )
            rdma.start()
            x_rdmas.append(rdma)
            dma = pltpu.make_async_copy(
                part_ref.at[c],
                out_ref.at[pl.ds(r, ch), :],
                copyout_sems.at[c],
            )
            dma.start()
            out_dmas.append(dma)

        for c in range(C):
            y_rdmas[c].wait_send()
            x_rdmas[c].wait_send()
            x_rdmas[c].wait_recv()
            out_dmas[c].wait()

    return pl.pallas_call(
        body,
        out_shape=jax.ShapeDtypeStruct((m, n), x.dtype),
        in_specs=[pl.BlockSpec(memory_space=pl.ANY)],
        out_specs=pl.BlockSpec(memory_space=pl.ANY),
        scratch_shapes=[
            pltpu.VMEM((C, ch, n), x.dtype),
            pltpu.VMEM((C, ch, n), x.dtype),
            pltpu.VMEM((C, ch, n), x.dtype),
            pltpu.SemaphoreType.DMA((C,)),
            pltpu.SemaphoreType.DMA((C,)),
            pltpu.SemaphoreType.DMA((C,)),
            pltpu.SemaphoreType.DMA((C,)),
            pltpu.SemaphoreType.DMA((C,)),
            pltpu.SemaphoreType.DMA((C,)),
        ],
        compiler_params=pltpu.CompilerParams(
            collective_id=0, vmem_limit_bytes=48 * 1024 * 1024
        ),
    )(x)


# device time: 114182 ns/iter; 1.0663x vs baseline; 1.0001x over previous
import jax
import jax.numpy as jnp
from jax import lax
from jax.experimental import pallas as pl
from jax.experimental.pallas import tpu as pltpu

C = 32


def kernel(x):
    _, m, n2 = x.shape
    n = n2 // 2
    half_m = m // 2
    ch = half_m // C

    def body(
        x_ref,
        out_ref,
        comm_ref,
        lin_ref,
        part_ref,
        y_send,
        y_recv,
        x_send,
        x_recv,
        lin_sems,
        copyout_sems,
    ):
        my_x = lax.axis_index("x")
        my_y = lax.axis_index("y")
        y_peer = (my_x, 1 - my_y)
        x_peer = (1 - my_x, my_y)

        barrier = pltpu.get_barrier_semaphore()
        for p in (y_peer, x_peer):
            pl.semaphore_signal(
                barrier, inc=1, device_id=p, device_id_type=pl.DeviceIdType.MESH
            )
        pl.semaphore_wait(barrier, 2)

        row0 = my_x * half_m
        my_col0 = my_y * n
        peer_col0 = (1 - my_y) * n

        y_rdmas = []
        x_rdmas = []
        lin_dmas = []
        out_dmas = []

        for c in range(C):
            r = row0 + c * ch
            dma = pltpu.make_async_copy(
                x_ref.at[0, pl.ds(r, ch), pl.ds(my_col0, n)],
                lin_ref.at[c],
                lin_sems.at[c],
            )
            dma.start()
            lin_dmas.append(dma)
            rdma = pltpu.make_async_remote_copy(
                src_ref=x_ref.at[0, pl.ds(r, ch), pl.ds(peer_col0, n)],
                dst_ref=comm_ref.at[c],
                send_sem=y_send.at[c],
                recv_sem=y_recv.at[c],
                device_id=y_peer,
                device_id_type=pl.DeviceIdType.MESH,
            )
            rdma.start()
            y_rdmas.append(rdma)

        for c in range(C):
            r = row0 + c * ch
            y_rdmas[c].wait_recv()
            lin_dmas[c].wait()
            part_ref[c] = lin_ref[c] + comm_ref[c]
            rdma = pltpu.make_async_remote_copy(
                src_ref=part_ref.at[c],
                dst_ref=out_ref.at[pl.ds(r, ch), :],
                send_sem=x_send.at[c],
                recv_sem=x_recv.at[c],
                device_id=x_peer,
                device_id_type=pl.DeviceIdType.MESH,
            )
            rdma.start()
            x_rdmas.append(rdma)
            dma = pltpu.make_async_copy(
                part_ref.at[c],
                out_ref.at[pl.ds(r, ch), :],
                copyout_sems.at[c],
            )
            dma.start()
            out_dmas.append(dma)

        for c in range(C):
            y_rdmas[c].wait_send()
            x_rdmas[c].wait_send()
            x_rdmas[c].wait_recv()
            out_dmas[c].wait()

    out = pl.pallas_call(
        body,
        out_shape=jax.ShapeDtypeStruct((m, n), x.dtype),
        in_specs=[pl.BlockSpec(memory_space=pl.ANY)],
        out_specs=pl.BlockSpec(memory_space=pl.ANY),
        scratch_shapes=[
            pltpu.VMEM((C, ch, n), x.dtype),
            pltpu.VMEM((C, ch, n), x.dtype),
            pltpu.VMEM((C, ch, n), x.dtype),
            pltpu.SemaphoreType.DMA((C,)),
            pltpu.SemaphoreType.DMA((C,)),
            pltpu.SemaphoreType.DMA((C,)),
            pltpu.SemaphoreType.DMA((C,)),
            pltpu.SemaphoreType.DMA((C,)),
            pltpu.SemaphoreType.DMA((C,)),
        ],
        compiler_params=pltpu.CompilerParams(
            collective_id=0, vmem_limit_bytes=48 * 1024 * 1024
        ),
    )(x)
    return out
